# Initial kernel scaffold; baseline (speedup 1.0000x reference)
import jax
import jax.numpy as jnp
from jax.experimental import pallas as pl


def kernel(node_neighbors, edge_neighbors, node_feats, edge_feats, W_edge, b_edge, W_edge_com, b_edge_com, W_node, b_node, W_node_com, b_node_com, ln_gamma, ln_beta):
    raise NotImplementedError("write your pallas kernel here")


# speedup vs baseline: 2.7831x; 2.7831x over previous
import functools

import jax
import jax.numpy as jnp
from jax import lax
from jax.experimental import pallas as pl
from jax.experimental.pallas import tpu as pltpu
from jax.experimental.pallas import tpu_sc as plsc

N = 10000
E = 320000
K_NODE = 32
K_EDGE = 8
D_NODE = 128
D_EDGE = 16
H = 128
EPS = 1e-5

NC = 2
NS = 16
NW = NC * NS
LANES = 16

N_PAD = 10240


def _layer_norm_rows(y, gamma, beta):
    mu = jnp.mean(y, axis=-1, keepdims=True)
    var = jnp.mean((y - mu) ** 2, axis=-1, keepdims=True)
    return (y - mu) / jnp.sqrt(var + EPS) * gamma + beta


def _edge_proj_body(x_ref, w_ref, b_ref, o_ref):
    o_ref[...] = (
        jnp.dot(x_ref[...], w_ref[...], preferred_element_type=jnp.float32)
        + b_ref[...]
    )


def _edge_proj(edge_feats, W_edge, b_edge):
    BE = 2000
    return pl.pallas_call(
        _edge_proj_body,
        grid=(E // BE,),
        in_specs=[
            pl.BlockSpec((BE, D_EDGE), lambda i: (i, 0)),
            pl.BlockSpec((D_EDGE, H), lambda i: (0, 0)),
            pl.BlockSpec((H,), lambda i: (0,)),
        ],
        out_specs=pl.BlockSpec((BE, H), lambda i: (i, 0)),
        out_shape=jax.ShapeDtypeStruct((E, H), jnp.float32),
    )(edge_feats, W_edge, b_edge)


def _edge_com_body(h_ref, m_ref, w_ref, b_ref, g_ref, bt_ref, o_ref):
    x = h_ref[...] + m_ref[...]
    y = jnp.dot(x, w_ref[...], preferred_element_type=jnp.float32) + b_ref[...]
    y = jnp.maximum(y, 0.0)
    o_ref[...] = _layer_norm_rows(y, g_ref[...], bt_ref[...])


def _edge_com(h_e, m_e, W_com, b_com, gamma, beta):
    BE = 2000
    return pl.pallas_call(
        _edge_com_body,
        grid=(E // BE,),
        in_specs=[
            pl.BlockSpec((BE, H), lambda i: (i, 0)),
            pl.BlockSpec((BE, H), lambda i: (i, 0)),
            pl.BlockSpec((H, H), lambda i: (0, 0)),
            pl.BlockSpec((H,), lambda i: (0,)),
            pl.BlockSpec((H,), lambda i: (0,)),
            pl.BlockSpec((H,), lambda i: (0,)),
        ],
        out_specs=pl.BlockSpec((BE, H), lambda i: (i, 0)),
        out_shape=jax.ShapeDtypeStruct((E, H), jnp.float32),
    )(h_e, m_e, W_com, b_com, gamma, beta)


def _node_body(x_ref, m_ref, w1_ref, b1_ref, w2_ref, b2_ref, g_ref, bt_ref,
               o_ref):
    h = jnp.dot(x_ref[...], w1_ref[...], preferred_element_type=jnp.float32)
    h = h + b1_ref[...] + m_ref[...]
    y = jnp.dot(h, w2_ref[...], preferred_element_type=jnp.float32) + b2_ref[...]
    y = jnp.maximum(y, 0.0)
    o_ref[...] = _layer_norm_rows(y, g_ref[...], bt_ref[...])


def _node_stage(node_feats_pad, m_n, W_node, b_node, W_com, b_com, gamma, beta):
    BN = 1024
    return pl.pallas_call(
        _node_body,
        grid=(N_PAD // BN,),
        in_specs=[
            pl.BlockSpec((BN, D_NODE), lambda i: (i, 0)),
            pl.BlockSpec((BN, H), lambda i: (i, 0)),
            pl.BlockSpec((D_NODE, H), lambda i: (0, 0)),
            pl.BlockSpec((H,), lambda i: (0,)),
            pl.BlockSpec((H, H), lambda i: (0, 0)),
            pl.BlockSpec((H,), lambda i: (0,)),
            pl.BlockSpec((H,), lambda i: (0,)),
            pl.BlockSpec((H,), lambda i: (0,)),
        ],
        out_specs=pl.BlockSpec((BN, H), lambda i: (i, 0)),
        out_shape=jax.ShapeDtypeStruct((N_PAD, H), jnp.float32),
    )(node_feats_pad, m_n, W_node, b_node, W_com, b_com, gamma, beta)


def _sc_gather_mean(table, idx1d, n_rows, K, C):
    G = (C * K) // 128
    assert C * K == G * 128
    rows_per_w = n_rows // NW
    nchunks = rows_per_w // C
    assert rows_per_w == nchunks * C
    inv_k = 1.0 / K

    mesh = plsc.VectorSubcoreMesh(core_axis_name="c", subcore_axis_name="s")

    @functools.partial(
        pl.kernel,
        out_type=jax.ShapeDtypeStruct((n_rows, H), jnp.float32),
        mesh=mesh,
        scratch_types=[
            pltpu.VMEM((C * K,), jnp.int32),
            pltpu.VMEM((G * 128, H), jnp.float32),
            pltpu.VMEM((C, H), jnp.float32),
            pltpu.SemaphoreType.DMA,
        ],
    )
    def k(table_hbm, idx_hbm, out_hbm, idx_v, nbr_v, out_v, sem):
        wid = lax.axis_index("s") * NC + lax.axis_index("c")

        def chunk_body(i, carry):
            cid = wid * nchunks + i
            row_base = cid * C
            pltpu.sync_copy(idx_hbm.at[pl.ds(cid * (C * K), C * K)], idx_v)
            cps = [
                pltpu.async_copy(
                    table_hbm.at[idx_v.at[pl.ds(g * 128, 128)]],
                    nbr_v.at[pl.ds(g * 128, 128)],
                    sem,
                )
                for g in range(G)
            ]
            for cp in cps:
                cp.wait()

            def row_body(c, carry2):
                base = c * K
                for d in range(H // LANES):
                    sl = pl.ds(d * LANES, LANES)
                    acc = nbr_v[base, sl]
                    for kk in range(1, K):
                        acc = acc + nbr_v[base + kk, sl]
                    out_v[c, sl] = acc * inv_k
                return carry2

            lax.fori_loop(0, C, row_body, 0)
            pltpu.sync_copy(out_v, out_hbm.at[pl.ds(row_base, C)])
            return carry

        lax.fori_loop(0, nchunks, chunk_body, 0)

    return k(table, idx1d)


def kernel(node_neighbors, edge_neighbors, node_feats, edge_feats,
           W_edge, b_edge, W_edge_com, b_edge_com,
           W_node, b_node, W_node_com, b_node_com,
           ln_gamma, ln_beta):
    edge_idx = jnp.reshape(edge_neighbors.astype(jnp.int32), (E * K_EDGE,))
    node_idx = jnp.pad(node_neighbors.astype(jnp.int32), ((0, N_PAD - N), (0, 0)))
    node_idx = jnp.reshape(node_idx, (N_PAD * K_NODE,))
    node_feats_pad = jnp.pad(node_feats, ((0, N_PAD - N), (0, 0)))

    h_e = _edge_proj(edge_feats, W_edge, b_edge)
    m_e = _sc_gather_mean(h_e, edge_idx, E, K_EDGE, C=80)
    new_edge = _edge_com(h_e, m_e, W_edge_com, b_edge_com,
                         ln_gamma, ln_beta)
    m_n = _sc_gather_mean(new_edge, node_idx, N_PAD, K_NODE, C=16)
    new_node_pad = _node_stage(node_feats_pad, m_n, W_node, b_node,
                               W_node_com, b_node_com, ln_gamma, ln_beta)
    return (new_node_pad[:N], new_edge)
